# baseline (speedup 1.0000x reference)
import jax
import jax.numpy as jnp
from jax.experimental import pallas as pl

_SHIFT = 8192


def _copy_body(in_ref, out_ref):
    out_ref[...] = in_ref[...]


def kernel(wav):
    s, b, c, length = wav.shape
    out_len = length - _SHIFT
    rows = s * b * c
    x = wav.reshape(rows, length)

    rows_per_block = 8
    out = pl.pallas_call(
        _copy_body,
        grid=(rows // rows_per_block,),
        in_specs=[pl.BlockSpec((rows_per_block, out_len), lambda i: (i, 0))],
        out_specs=pl.BlockSpec((rows_per_block, out_len), lambda i: (i, 0)),
        out_shape=jax.ShapeDtypeStruct((rows, out_len), wav.dtype),
    )(x)
    return out.reshape(s, b, c, out_len)
